# baseline (speedup 1.0000x reference)
import functools

import jax
import jax.numpy as jnp
from jax import lax
from jax.experimental import pallas as pl
from jax.experimental.pallas import tpu as pltpu
from jax.experimental.pallas import tpu_sc as plsc

N = 10000
E = 320000
D = 128

NC = 2
NS = 16
NT = NC * NS
CH = 16
NB = 16
NCH = 625
PE = NT * NCH * CH
NPAD = 10240
RPT = NPAD // NS
BLK = 1024
SHIFT = 14
MASK = (1 << SHIFT) - 1


def _sc_mesh():
    return plsc.VectorSubcoreMesh(core_axis_name="c", subcore_axis_name="s")


@functools.partial(
    pl.kernel,
    out_type=jax.ShapeDtypeStruct((NC, NPAD), jnp.float32),
    mesh=_sc_mesh(),
    scratch_types=[
        pltpu.VMEM((NCH * CH,), jnp.int32),
        pltpu.VMEM((1, CH), jnp.int32),
        pltpu.VMEM((CH,), jnp.float32),
        pltpu.VMEM_SHARED((NPAD,), jnp.float32),
    ],
)
def _sc_degree(pk_hbm, zrow_hbm, ones_hbm, deg_out, pidx, dstage, ones_v, acc):
    cid = lax.axis_index("c")
    sid = lax.axis_index("s")
    wid = cid * NS + sid
    pltpu.sync_copy(zrow_hbm, acc.at[pl.ds(sid * RPT, RPT)])
    pltpu.sync_copy(pk_hbm.at[wid], pidx)
    pltpu.sync_copy(ones_hbm, ones_v)
    plsc.subcore_barrier()

    def body(c, carry):
        for j in range(CH // 16):
            p = pidx[pl.ds(c * CH + j * 16, 16)]
            dstage[0, pl.ds(j * 16, 16)] = jnp.right_shift(p, SHIFT)
        pltpu.sync_copy(ones_v, acc.at[dstage.at[0]], add=True)
        return carry

    lax.fori_loop(0, NCH, body, 0)
    plsc.subcore_barrier()
    pltpu.sync_copy(acc.at[pl.ds(sid * RPT, RPT)],
                    deg_out.at[cid, pl.ds(sid * RPT, RPT)])


@functools.partial(
    pl.kernel,
    out_type=jax.ShapeDtypeStruct((NC, NPAD, D), jnp.float32),
    mesh=_sc_mesh(),
    scratch_types=[
        pltpu.VMEM((NCH * CH,), jnp.int32),
        pltpu.VMEM((NB, CH), jnp.int32),
        pltpu.VMEM((NB, CH), jnp.int32),
        pltpu.VMEM((NB, CH, D), jnp.float32),
        pltpu.VMEM_SHARED((NPAD, D), jnp.float32),
        pltpu.SemaphoreType.DMA((NB,)),
    ],
)
def _sc_scatter(pk_hbm, y_hbm, zmat_hbm, out_hbm,
                pidx, gring, dring, rows, acc, sem):
    cid = lax.axis_index("c")
    sid = lax.axis_index("s")
    wid = cid * NS + sid
    pltpu.sync_copy(zmat_hbm, acc.at[pl.ds(sid * RPT, RPT)])
    pltpu.sync_copy(pk_hbm.at[wid], pidx)
    plsc.subcore_barrier()

    def unpack(c, b):
        for j in range(CH // 16):
            p = pidx[pl.ds(c * CH + j * 16, 16)]
            gring[b, pl.ds(j * 16, 16)] = jnp.bitwise_and(p, MASK)
            dring[b, pl.ds(j * 16, 16)] = jnp.right_shift(p, SHIFT)

    for b in range(NB):
        unpack(b, b)
        pltpu.async_copy(y_hbm.at[gring.at[b]], rows.at[b], sem.at[b])

    def body(c, carry):
        b = lax.rem(c, NB)
        pltpu.make_async_copy(y_hbm.at[gring.at[b]], rows.at[b],
                              sem.at[b]).wait()
        pltpu.sync_copy(rows.at[b], acc.at[dring.at[b]], add=True)

        @pl.when(c + NB < NCH)
        def _():
            unpack(c + NB, b)
            pltpu.async_copy(y_hbm.at[gring.at[b]], rows.at[b], sem.at[b])

        return carry

    lax.fori_loop(0, NCH, body, 0)
    plsc.subcore_barrier()
    pltpu.sync_copy(acc.at[pl.ds(sid * RPT, RPT)],
                    out_hbm.at[cid, pl.ds(sid * RPT, RPT)])


def _tc1_body(x_ref, w_ref, deg_ref, y_ref, dinv_ref):
    deg = deg_ref[0, :] + deg_ref[1, :] + 1.0
    dinv = lax.rsqrt(jnp.maximum(deg, 1.0))
    xw = jnp.dot(x_ref[...], w_ref[...], preferred_element_type=jnp.float32)
    y_ref[...] = xw * dinv[:, None]
    dinv_ref[...] = dinv[:, None]


_tc1 = pl.pallas_call(
    _tc1_body,
    grid=(NPAD // BLK,),
    in_specs=[
        pl.BlockSpec((BLK, D), lambda i: (i, 0)),
        pl.BlockSpec((D, D), lambda i: (0, 0)),
        pl.BlockSpec((NC, BLK), lambda i: (0, i)),
    ],
    out_specs=[
        pl.BlockSpec((BLK, D), lambda i: (i, 0)),
        pl.BlockSpec((BLK, 1), lambda i: (i, 0)),
    ],
    out_shape=[
        jax.ShapeDtypeStruct((NPAD, D), jnp.float32),
        jax.ShapeDtypeStruct((NPAD, 1), jnp.float32),
    ],
)


def _tc2_body(p_ref, y1_ref, dinv_ref, b_ref, w_ref, y2_ref):
    i = pl.program_id(0)
    s = p_ref[0] + p_ref[1] + y1_ref[...]
    pre = s * dinv_ref[...] + b_ref[...]
    rows = lax.broadcasted_iota(jnp.int32, (BLK, D), 0) + i * BLK
    h = jnp.where(rows < N, jnp.maximum(pre, 0.0), 0.0)
    y2_ref[...] = (
        jnp.dot(h, w_ref[...], preferred_element_type=jnp.float32)
        * dinv_ref[...]
    )


_tc2 = pl.pallas_call(
    _tc2_body,
    grid=(NPAD // BLK,),
    in_specs=[
        pl.BlockSpec((NC, BLK, D), lambda i: (0, i, 0)),
        pl.BlockSpec((BLK, D), lambda i: (i, 0)),
        pl.BlockSpec((BLK, 1), lambda i: (i, 0)),
        pl.BlockSpec((1, D), lambda i: (0, 0)),
        pl.BlockSpec((D, D), lambda i: (0, 0)),
    ],
    out_specs=pl.BlockSpec((BLK, D), lambda i: (i, 0)),
    out_shape=jax.ShapeDtypeStruct((NPAD, D), jnp.float32),
)


def _tc3_body(p_ref, y2_ref, dinv_ref, b_ref, o_ref):
    o = (p_ref[0] + p_ref[1] + y2_ref[...]) * dinv_ref[...] + b_ref[...]
    m = jnp.max(o, axis=1, keepdims=True)
    z = o - m
    lse = jnp.log(jnp.sum(jnp.exp(z), axis=1, keepdims=True))
    o_ref[...] = z - lse


_tc3 = pl.pallas_call(
    _tc3_body,
    grid=(NPAD // BLK,),
    in_specs=[
        pl.BlockSpec((NC, BLK, D), lambda i: (0, i, 0)),
        pl.BlockSpec((BLK, D), lambda i: (i, 0)),
        pl.BlockSpec((BLK, 1), lambda i: (i, 0)),
        pl.BlockSpec((1, D), lambda i: (0, 0)),
    ],
    out_specs=pl.BlockSpec((BLK, D), lambda i: (i, 0)),
    out_shape=jax.ShapeDtypeStruct((NPAD, D), jnp.float32),
)


def kernel(x, edge_index, W1, b1, W2, b2):
    x = x.astype(jnp.float32)
    xpad = jnp.zeros((NPAD, D), jnp.float32).at[:N].set(x)
    pad = PE - E
    padv = jnp.full((pad,), N | (N << SHIFT), jnp.int32)
    packed = jnp.bitwise_or(edge_index[0],
                            jnp.left_shift(edge_index[1], SHIFT))
    packed = jnp.concatenate([packed, padv]).reshape(NT, NCH * CH)
    zrow = jnp.zeros((RPT,), jnp.float32)
    zmat = jnp.zeros((RPT, D), jnp.float32)
    ones = jnp.ones((CH,), jnp.float32)

    degp = _sc_degree(packed, zrow, ones)
    y1, dinv = _tc1(xpad, W1, degp)
    parts1 = _sc_scatter(packed, y1, zmat)
    y2 = _tc2(parts1, y1, dinv, b1.reshape(1, D), W2)
    parts2 = _sc_scatter(packed, y2, zmat)
    out = _tc3(parts2, y2, dinv, b2.reshape(1, D))
    return out[:N]


# speedup vs baseline: 1.0125x; 1.0125x over previous
import functools

import jax
import jax.numpy as jnp
from jax import lax
from jax.experimental import pallas as pl
from jax.experimental.pallas import tpu as pltpu
from jax.experimental.pallas import tpu_sc as plsc

N = 10000
E = 320000
D = 128

NC = 2
NS = 16
NT = NC * NS
CH = 32
NB = 8
NCH = 313
PE = NT * NCH * CH
NPAD = 10240
RPT = NPAD // NS
BLK = 1024
SHIFT = 14
MASK = (1 << SHIFT) - 1


def _sc_mesh():
    return plsc.VectorSubcoreMesh(core_axis_name="c", subcore_axis_name="s")


@functools.partial(
    pl.kernel,
    out_type=jax.ShapeDtypeStruct((NC, NPAD), jnp.float32),
    mesh=_sc_mesh(),
    scratch_types=[
        pltpu.VMEM((NCH * CH,), jnp.int32),
        pltpu.VMEM((1, CH), jnp.int32),
        pltpu.VMEM((CH,), jnp.float32),
        pltpu.VMEM_SHARED((NPAD,), jnp.float32),
    ],
)
def _sc_degree(pk_hbm, zrow_hbm, ones_hbm, deg_out, pidx, dstage, ones_v, acc):
    cid = lax.axis_index("c")
    sid = lax.axis_index("s")
    wid = cid * NS + sid
    pltpu.sync_copy(zrow_hbm, acc.at[pl.ds(sid * RPT, RPT)])
    pltpu.sync_copy(pk_hbm.at[wid], pidx)
    pltpu.sync_copy(ones_hbm, ones_v)
    plsc.subcore_barrier()

    def body(c, carry):
        for j in range(CH // 16):
            p = pidx[pl.ds(c * CH + j * 16, 16)]
            dstage[0, pl.ds(j * 16, 16)] = jnp.right_shift(p, SHIFT)
        pltpu.sync_copy(ones_v, acc.at[dstage.at[0]], add=True)
        return carry

    lax.fori_loop(0, NCH, body, 0)
    plsc.subcore_barrier()
    pltpu.sync_copy(acc.at[pl.ds(sid * RPT, RPT)],
                    deg_out.at[cid, pl.ds(sid * RPT, RPT)])


@functools.partial(
    pl.kernel,
    out_type=jax.ShapeDtypeStruct((NC, NPAD, D), jnp.float32),
    mesh=_sc_mesh(),
    scratch_types=[
        pltpu.VMEM((NCH * CH,), jnp.int32),
        pltpu.VMEM((NB, CH), jnp.int32),
        pltpu.VMEM((NB, CH), jnp.int32),
        pltpu.VMEM((NB, CH, D), jnp.float32),
        pltpu.VMEM_SHARED((NPAD, D), jnp.float32),
        pltpu.SemaphoreType.DMA((NB,)),
    ],
)
def _sc_scatter(pk_hbm, y_hbm, zmat_hbm, out_hbm,
                pidx, gring, dring, rows, acc, sem):
    cid = lax.axis_index("c")
    sid = lax.axis_index("s")
    wid = cid * NS + sid
    pltpu.sync_copy(zmat_hbm, acc.at[pl.ds(sid * RPT, RPT)])
    pltpu.sync_copy(pk_hbm.at[wid], pidx)
    plsc.subcore_barrier()

    def unpack(c, b):
        for j in range(CH // 16):
            p = pidx[pl.ds(c * CH + j * 16, 16)]
            gring[b, pl.ds(j * 16, 16)] = jnp.bitwise_and(p, MASK)
            dring[b, pl.ds(j * 16, 16)] = jnp.right_shift(p, SHIFT)

    for b in range(NB):
        unpack(b, b)
        pltpu.async_copy(y_hbm.at[gring.at[b]], rows.at[b], sem.at[b])

    def body(c, carry):
        b = lax.rem(c, NB)
        pltpu.make_async_copy(y_hbm.at[gring.at[b]], rows.at[b],
                              sem.at[b]).wait()
        pltpu.sync_copy(rows.at[b], acc.at[dring.at[b]], add=True)

        @pl.when(c + NB < NCH)
        def _():
            unpack(c + NB, b)
            pltpu.async_copy(y_hbm.at[gring.at[b]], rows.at[b], sem.at[b])

        return carry

    lax.fori_loop(0, NCH, body, 0)
    plsc.subcore_barrier()
    pltpu.sync_copy(acc.at[pl.ds(sid * RPT, RPT)],
                    out_hbm.at[cid, pl.ds(sid * RPT, RPT)])


def _tc1_body(x_ref, w_ref, deg_ref, y_ref, dinv_ref):
    deg = deg_ref[0, :] + deg_ref[1, :] + 1.0
    dinv = lax.rsqrt(jnp.maximum(deg, 1.0))
    xw = jnp.dot(x_ref[...], w_ref[...], preferred_element_type=jnp.float32)
    y_ref[...] = xw * dinv[:, None]
    dinv_ref[...] = dinv[:, None]


_tc1 = pl.pallas_call(
    _tc1_body,
    grid=(NPAD // BLK,),
    in_specs=[
        pl.BlockSpec((BLK, D), lambda i: (i, 0)),
        pl.BlockSpec((D, D), lambda i: (0, 0)),
        pl.BlockSpec((NC, BLK), lambda i: (0, i)),
    ],
    out_specs=[
        pl.BlockSpec((BLK, D), lambda i: (i, 0)),
        pl.BlockSpec((BLK, 1), lambda i: (i, 0)),
    ],
    out_shape=[
        jax.ShapeDtypeStruct((NPAD, D), jnp.float32),
        jax.ShapeDtypeStruct((NPAD, 1), jnp.float32),
    ],
)


def _tc2_body(p_ref, y1_ref, dinv_ref, b_ref, w_ref, y2_ref):
    i = pl.program_id(0)
    s = p_ref[0] + p_ref[1] + y1_ref[...]
    pre = s * dinv_ref[...] + b_ref[...]
    rows = lax.broadcasted_iota(jnp.int32, (BLK, D), 0) + i * BLK
    h = jnp.where(rows < N, jnp.maximum(pre, 0.0), 0.0)
    y2_ref[...] = (
        jnp.dot(h, w_ref[...], preferred_element_type=jnp.float32)
        * dinv_ref[...]
    )


_tc2 = pl.pallas_call(
    _tc2_body,
    grid=(NPAD // BLK,),
    in_specs=[
        pl.BlockSpec((NC, BLK, D), lambda i: (0, i, 0)),
        pl.BlockSpec((BLK, D), lambda i: (i, 0)),
        pl.BlockSpec((BLK, 1), lambda i: (i, 0)),
        pl.BlockSpec((1, D), lambda i: (0, 0)),
        pl.BlockSpec((D, D), lambda i: (0, 0)),
    ],
    out_specs=pl.BlockSpec((BLK, D), lambda i: (i, 0)),
    out_shape=jax.ShapeDtypeStruct((NPAD, D), jnp.float32),
)


def _tc3_body(p_ref, y2_ref, dinv_ref, b_ref, o_ref):
    o = (p_ref[0] + p_ref[1] + y2_ref[...]) * dinv_ref[...] + b_ref[...]
    m = jnp.max(o, axis=1, keepdims=True)
    z = o - m
    lse = jnp.log(jnp.sum(jnp.exp(z), axis=1, keepdims=True))
    o_ref[...] = z - lse


_tc3 = pl.pallas_call(
    _tc3_body,
    grid=(NPAD // BLK,),
    in_specs=[
        pl.BlockSpec((NC, BLK, D), lambda i: (0, i, 0)),
        pl.BlockSpec((BLK, D), lambda i: (i, 0)),
        pl.BlockSpec((BLK, 1), lambda i: (i, 0)),
        pl.BlockSpec((1, D), lambda i: (0, 0)),
    ],
    out_specs=pl.BlockSpec((BLK, D), lambda i: (i, 0)),
    out_shape=jax.ShapeDtypeStruct((NPAD, D), jnp.float32),
)


def kernel(x, edge_index, W1, b1, W2, b2):
    x = x.astype(jnp.float32)
    xpad = jnp.zeros((NPAD, D), jnp.float32).at[:N].set(x)
    pad = PE - E
    padv = jnp.full((pad,), N | (N << SHIFT), jnp.int32)
    packed = jnp.bitwise_or(edge_index[0],
                            jnp.left_shift(edge_index[1], SHIFT))
    packed = jnp.concatenate([packed, padv]).reshape(NT, NCH * CH)
    zrow = jnp.zeros((RPT,), jnp.float32)
    zmat = jnp.zeros((RPT, D), jnp.float32)
    ones = jnp.ones((CH,), jnp.float32)

    degp = _sc_degree(packed, zrow, ones)
    y1, dinv = _tc1(xpad, W1, degp)
    parts1 = _sc_scatter(packed, y1, zmat)
    y2 = _tc2(parts1, y1, dinv, b1.reshape(1, D), W2)
    parts2 = _sc_scatter(packed, y2, zmat)
    out = _tc3(parts2, y2, dinv, b2.reshape(1, D))
    return out[:N]


# speedup vs baseline: 1.0796x; 1.0663x over previous
import functools

import jax
import jax.numpy as jnp
from jax import lax
from jax.experimental import pallas as pl
from jax.experimental.pallas import tpu as pltpu
from jax.experimental.pallas import tpu_sc as plsc

N = 10000
E = 320000
D = 128

NC = 2
NS = 16
NT = NC * NS
CH = 32
NB = 8
NCH = 313
PE = NT * NCH * CH
NPAD = 10240
RPT = NPAD // NS
BLK = 1024
SHIFT = 14
MASK = (1 << SHIFT) - 1


def _sc_mesh():
    return plsc.VectorSubcoreMesh(core_axis_name="c", subcore_axis_name="s")


@functools.partial(
    pl.kernel,
    out_type=jax.ShapeDtypeStruct((NC, NPAD), jnp.float32),
    mesh=_sc_mesh(),
    scratch_types=[
        pltpu.VMEM((NCH * CH,), jnp.int32),
        pltpu.VMEM((8, CH), jnp.int32),
        pltpu.VMEM((CH,), jnp.float32),
        pltpu.VMEM_SHARED((NPAD,), jnp.float32),
        pltpu.SemaphoreType.DMA((8,)),
    ],
)
def _sc_degree(pk_hbm, zrow_hbm, ones_hbm, deg_out, pidx, dstage, ones_v, acc,
               dsem):
    cid = lax.axis_index("c")
    sid = lax.axis_index("s")
    wid = cid * NS + sid
    pltpu.sync_copy(zrow_hbm, acc.at[pl.ds(sid * RPT, RPT)])
    pltpu.sync_copy(pk_hbm.at[wid], pidx)
    pltpu.sync_copy(ones_hbm, ones_v)
    plsc.subcore_barrier()

    def body(c, carry):
        b = lax.rem(c, 8)

        @pl.when(c >= 8)
        def _():
            pltpu.make_async_copy(ones_v, acc.at[dstage.at[b]],
                                  dsem.at[b]).wait()

        for j in range(CH // 16):
            p = pidx[pl.ds(c * CH + j * 16, 16)]
            dstage[b, pl.ds(j * 16, 16)] = jnp.right_shift(p, SHIFT)
        pltpu.async_copy(ones_v, acc.at[dstage.at[b]], dsem.at[b],
                         add=True)
        return carry

    lax.fori_loop(0, NCH, body, 0)
    for b in range(8):
        pltpu.make_async_copy(ones_v, acc.at[dstage.at[b]], dsem.at[b]).wait()
    plsc.subcore_barrier()
    pltpu.sync_copy(acc.at[pl.ds(sid * RPT, RPT)],
                    deg_out.at[cid, pl.ds(sid * RPT, RPT)])


@functools.partial(
    pl.kernel,
    out_type=jax.ShapeDtypeStruct((NC, NPAD, D), jnp.float32),
    mesh=_sc_mesh(),
    scratch_types=[
        pltpu.VMEM((NCH * CH,), jnp.int32),
        pltpu.VMEM((NB, CH), jnp.int32),
        pltpu.VMEM((NB, CH), jnp.int32),
        pltpu.VMEM((NB, CH, D), jnp.float32),
        pltpu.VMEM_SHARED((NPAD, D), jnp.float32),
        pltpu.SemaphoreType.DMA((NB,)),
    ],
)
def _sc_scatter(pk_hbm, y_hbm, zmat_hbm, out_hbm,
                pidx, gring, dring, rows, acc, sem):
    cid = lax.axis_index("c")
    sid = lax.axis_index("s")
    wid = cid * NS + sid
    pltpu.sync_copy(zmat_hbm, acc.at[pl.ds(sid * RPT, RPT)])
    pltpu.sync_copy(pk_hbm.at[wid], pidx)
    plsc.subcore_barrier()

    def unpack(c, b):
        for j in range(CH // 16):
            p = pidx[pl.ds(c * CH + j * 16, 16)]
            gring[b, pl.ds(j * 16, 16)] = jnp.bitwise_and(p, MASK)
            dring[b, pl.ds(j * 16, 16)] = jnp.right_shift(p, SHIFT)

    for b in range(NB):
        unpack(b, b)
        pltpu.async_copy(y_hbm.at[gring.at[b]], rows.at[b], sem.at[b])

    def body(c, carry):
        b = lax.rem(c, NB)
        pltpu.make_async_copy(y_hbm.at[gring.at[b]], rows.at[b],
                              sem.at[b]).wait()
        pltpu.sync_copy(rows.at[b], acc.at[dring.at[b]], add=True)

        @pl.when(c + NB < NCH)
        def _():
            unpack(c + NB, b)
            pltpu.async_copy(y_hbm.at[gring.at[b]], rows.at[b], sem.at[b])

        return carry

    lax.fori_loop(0, NCH, body, 0)
    plsc.subcore_barrier()
    pltpu.sync_copy(acc.at[pl.ds(sid * RPT, RPT)],
                    out_hbm.at[cid, pl.ds(sid * RPT, RPT)])


def _tc1_body(x_ref, w_ref, deg_ref, y_ref, dinv_ref):
    deg = deg_ref[0, :] + deg_ref[1, :] + 1.0
    dinv = lax.rsqrt(jnp.maximum(deg, 1.0))
    xw = jnp.dot(x_ref[...], w_ref[...], preferred_element_type=jnp.float32)
    y_ref[...] = xw * dinv[:, None]
    dinv_ref[...] = dinv[:, None]


_tc1 = pl.pallas_call(
    _tc1_body,
    grid=(NPAD // BLK,),
    in_specs=[
        pl.BlockSpec((BLK, D), lambda i: (i, 0)),
        pl.BlockSpec((D, D), lambda i: (0, 0)),
        pl.BlockSpec((NC, BLK), lambda i: (0, i)),
    ],
    out_specs=[
        pl.BlockSpec((BLK, D), lambda i: (i, 0)),
        pl.BlockSpec((BLK, 1), lambda i: (i, 0)),
    ],
    out_shape=[
        jax.ShapeDtypeStruct((NPAD, D), jnp.float32),
        jax.ShapeDtypeStruct((NPAD, 1), jnp.float32),
    ],
)


def _tc2_body(p_ref, y1_ref, dinv_ref, b_ref, w_ref, y2_ref):
    i = pl.program_id(0)
    s = p_ref[0] + p_ref[1] + y1_ref[...]
    pre = s * dinv_ref[...] + b_ref[...]
    rows = lax.broadcasted_iota(jnp.int32, (BLK, D), 0) + i * BLK
    h = jnp.where(rows < N, jnp.maximum(pre, 0.0), 0.0)
    y2_ref[...] = (
        jnp.dot(h, w_ref[...], preferred_element_type=jnp.float32)
        * dinv_ref[...]
    )


_tc2 = pl.pallas_call(
    _tc2_body,
    grid=(NPAD // BLK,),
    in_specs=[
        pl.BlockSpec((NC, BLK, D), lambda i: (0, i, 0)),
        pl.BlockSpec((BLK, D), lambda i: (i, 0)),
        pl.BlockSpec((BLK, 1), lambda i: (i, 0)),
        pl.BlockSpec((1, D), lambda i: (0, 0)),
        pl.BlockSpec((D, D), lambda i: (0, 0)),
    ],
    out_specs=pl.BlockSpec((BLK, D), lambda i: (i, 0)),
    out_shape=jax.ShapeDtypeStruct((NPAD, D), jnp.float32),
)


def _tc3_body(p_ref, y2_ref, dinv_ref, b_ref, o_ref):
    o = (p_ref[0] + p_ref[1] + y2_ref[...]) * dinv_ref[...] + b_ref[...]
    m = jnp.max(o, axis=1, keepdims=True)
    z = o - m
    lse = jnp.log(jnp.sum(jnp.exp(z), axis=1, keepdims=True))
    o_ref[...] = z - lse


_tc3 = pl.pallas_call(
    _tc3_body,
    grid=(NPAD // BLK,),
    in_specs=[
        pl.BlockSpec((NC, BLK, D), lambda i: (0, i, 0)),
        pl.BlockSpec((BLK, D), lambda i: (i, 0)),
        pl.BlockSpec((BLK, 1), lambda i: (i, 0)),
        pl.BlockSpec((1, D), lambda i: (0, 0)),
    ],
    out_specs=pl.BlockSpec((BLK, D), lambda i: (i, 0)),
    out_shape=jax.ShapeDtypeStruct((NPAD, D), jnp.float32),
)


def kernel(x, edge_index, W1, b1, W2, b2):
    x = x.astype(jnp.float32)
    xpad = jnp.zeros((NPAD, D), jnp.float32).at[:N].set(x)
    pad = PE - E
    padv = jnp.full((pad,), N | (N << SHIFT), jnp.int32)
    packed = jnp.bitwise_or(edge_index[0],
                            jnp.left_shift(edge_index[1], SHIFT))
    packed = jnp.concatenate([packed, padv]).reshape(NT, NCH * CH)
    zrow = jnp.zeros((RPT,), jnp.float32)
    zmat = jnp.zeros((RPT, D), jnp.float32)
    ones = jnp.ones((CH,), jnp.float32)

    degp = _sc_degree(packed, zrow, ones)
    y1, dinv = _tc1(xpad, W1, degp)
    parts1 = _sc_scatter(packed, y1, zmat)
    y2 = _tc2(parts1, y1, dinv, b1.reshape(1, D), W2)
    parts2 = _sc_scatter(packed, y2, zmat)
    out = _tc3(parts2, y2, dinv, b2.reshape(1, D))
    return out[:N]
